# Initial kernel scaffold; baseline (speedup 1.0000x reference)
import jax
import jax.numpy as jnp
from jax.experimental import pallas as pl


def kernel(online_x, target_x, W_enc, b_enc, W_pred, b_pred, codebook, W_enc_t, b_enc_t):
    raise NotImplementedError("write your pallas kernel here")


# speedup vs baseline: 1.0005x; 1.0005x over previous
import functools

import jax
import jax.numpy as jnp
from jax import lax
from jax.experimental import pallas as pl
from jax.experimental.pallas import tpu as pltpu
from jax.experimental.pallas import tpu_sc as plsc

_N = 100000
_IN_DIM = 128
_CODE_DIM = 64
_K = 1024
_COMMIT_W = 1.0

_BN = 1000
_NBLK = _N // _BN
_NW = 32
_PAD_N = 102400
_B_PER_W = _PAD_N // _NW
_CHUNK = 128
_NCHUNK = _B_PER_W // _CHUNK


def _tc_body(xo_ref, xt_ref, we_ref, be_ref, wp_ref, bp_ref, cb_ref, cbt_ref,
             wet_ref, bet_ref, idxo_ref, idxt_ref, cwb_ref, loss_ref):
    i = pl.program_id(0)
    cbt = cbt_ref[...]
    c2 = jnp.sum(cbt * cbt, axis=0, keepdims=True)

    @pl.when(i == 0)
    def _init():
        cwb_ref[...] = (
            jnp.dot(cb_ref[...], wp_ref[...], preferred_element_type=jnp.float32)
            + bp_ref[...]
        )
        loss_ref[...] = jnp.zeros_like(loss_ref)

    y = (
        jnp.dot(xo_ref[...], we_ref[...], preferred_element_type=jnp.float32)
        + be_ref[...]
    )
    s = jnp.dot(y, cbt, preferred_element_type=jnp.float32)
    score = c2 - 2.0 * s
    minv = jnp.min(score, axis=1, keepdims=True)
    iota = lax.broadcasted_iota(jnp.int32, score.shape, 1)
    idx = jnp.min(jnp.where(score == minv, iota, _K), axis=1)
    idxo_ref[0, 0, :] = idx
    x2 = jnp.sum(y * y, axis=1, keepdims=True)
    loss_ref[...] = loss_ref[...] + jnp.sum(x2 + minv)

    yt = (
        jnp.dot(xt_ref[...], wet_ref[...], preferred_element_type=jnp.float32)
        + bet_ref[...]
    )
    st = jnp.dot(yt, cbt, preferred_element_type=jnp.float32)
    scoret = c2 - 2.0 * st
    minvt = jnp.min(scoret, axis=1, keepdims=True)
    idxt = jnp.min(jnp.where(scoret == minvt, iota, _K), axis=1)
    idxt_ref[0, 0, :] = idxt


def _tc_forward(online_x, target_x, W_enc, b_enc, W_pred, b_pred, codebook,
                cbT, W_enc_t, b_enc_t):
    full = lambda shape: pl.BlockSpec(shape, lambda i: (0,) * len(shape))
    return pl.pallas_call(
        _tc_body,
        grid=(_NBLK,),
        in_specs=[
            pl.BlockSpec((_BN, _IN_DIM), lambda i: (i, 0)),
            pl.BlockSpec((_BN, _IN_DIM), lambda i: (i, 0)),
            full((_IN_DIM, _CODE_DIM)),
            full((1, _CODE_DIM)),
            full((_CODE_DIM, _CODE_DIM)),
            full((1, _CODE_DIM)),
            full((_K, _CODE_DIM)),
            full((_CODE_DIM, _K)),
            full((_IN_DIM, _CODE_DIM)),
            full((1, _CODE_DIM)),
        ],
        out_specs=[
            pl.BlockSpec((1, 1, _BN), lambda i: (i, 0, 0)),
            pl.BlockSpec((1, 1, _BN), lambda i: (i, 0, 0)),
            full((_K, _CODE_DIM)),
            full((1, 1)),
        ],
        out_shape=[
            jax.ShapeDtypeStruct((_NBLK, 1, _BN), jnp.int32),
            jax.ShapeDtypeStruct((_NBLK, 1, _BN), jnp.int32),
            jax.ShapeDtypeStruct((_K, _CODE_DIM), jnp.float32),
            jax.ShapeDtypeStruct((1, 1), jnp.float32),
        ],
        compiler_params=pltpu.CompilerParams(
            dimension_semantics=("arbitrary",),
        ),
    )(online_x, target_x, W_enc, b_enc, W_pred, b_pred, codebook, cbT,
      W_enc_t, b_enc_t)


def _sc_body(cwb_hbm, cb_hbm, idxo_hbm, idxt_hbm, outq_hbm, outt_hbm,
             idx1_v, rows1_v, idx2_v, rows2_v, sem1, sem2):
    wid = lax.axis_index("s") * 2 + lax.axis_index("c")
    base = wid * _B_PER_W

    def body(j, carry):
        off = base + j * _CHUNK
        pltpu.sync_copy(idxo_hbm.at[pl.ds(off, _CHUNK)], idx1_v)
        pltpu.sync_copy(idxt_hbm.at[pl.ds(off, _CHUNK)], idx2_v)
        cp1 = pltpu.async_copy(cwb_hbm.at[idx1_v], rows1_v, sem1)
        cp2 = pltpu.async_copy(cb_hbm.at[idx2_v], rows2_v, sem2)
        cp1.wait()
        cp2.wait()
        pltpu.sync_copy(rows1_v, outq_hbm.at[pl.ds(off, _CHUNK)])
        pltpu.sync_copy(rows2_v, outt_hbm.at[pl.ds(off, _CHUNK)])
        return carry

    lax.fori_loop(0, _NCHUNK, body, 0)


def _sc_gather(cwb, codebook, idxo_p, idxt_p):
    mesh = plsc.VectorSubcoreMesh(core_axis_name="c", subcore_axis_name="s")
    fn = functools.partial(
        pl.kernel,
        mesh=mesh,
        out_type=[
            jax.ShapeDtypeStruct((_PAD_N, _CODE_DIM), jnp.float32),
            jax.ShapeDtypeStruct((_PAD_N, _CODE_DIM), jnp.float32),
        ],
        scratch_types=[
            pltpu.VMEM((_CHUNK,), jnp.int32),
            pltpu.VMEM((_CHUNK, _CODE_DIM), jnp.float32),
            pltpu.VMEM((_CHUNK,), jnp.int32),
            pltpu.VMEM((_CHUNK, _CODE_DIM), jnp.float32),
            pltpu.SemaphoreType.DMA,
            pltpu.SemaphoreType.DMA,
        ],
        compiler_params=pltpu.CompilerParams(use_tc_tiling_on_sc=False),
    )(_sc_body)
    return fn(cwb, codebook, idxo_p, idxt_p)


def kernel(online_x, target_x, W_enc, b_enc, W_pred, b_pred, codebook,
           W_enc_t, b_enc_t):
    cbT = codebook.T
    idxo3, idxt3, cwb, loss = _tc_forward(
        online_x, target_x, W_enc, b_enc.reshape(1, -1), W_pred,
        b_pred.reshape(1, -1), codebook, cbT, W_enc_t, b_enc_t.reshape(1, -1))
    pad = _PAD_N - _N
    idxo_p = jnp.concatenate([idxo3.reshape(-1), jnp.zeros((pad,), jnp.int32)])
    idxt_p = jnp.concatenate([idxt3.reshape(-1), jnp.zeros((pad,), jnp.int32)])
    outq_p, outt_p = _sc_gather(cwb, codebook, idxo_p, idxt_p)
    online_q = outq_p[:_N]
    quantized_target = outt_p[:_N]
    commit_loss = loss[0, 0] * (_COMMIT_W / (_N * _CODE_DIM))
    return (online_q, quantized_target, commit_loss)


# speedup vs baseline: 1.0007x; 1.0002x over previous
import functools

import jax
import jax.numpy as jnp
from jax import lax
from jax.experimental import pallas as pl
from jax.experimental.pallas import tpu as pltpu
from jax.experimental.pallas import tpu_sc as plsc

_N = 100000
_IN_DIM = 128
_CODE_DIM = 64
_K = 1024
_COMMIT_W = 1.0

_BN = 1000
_NBLK = _N // _BN
_NW = 32
_PAD_N = 102400
_B_PER_W = _PAD_N // _NW
_CHUNK = 128
_NCHUNK = _B_PER_W // _CHUNK


def _tc_body(xo_ref, xt_ref, we_ref, be_ref, wp_ref, bp_ref, cb_ref, cbt_ref,
             wet_ref, bet_ref, idxo_ref, idxt_ref, cwb_ref, loss_ref):
    i = pl.program_id(0)
    cbt = cbt_ref[...]
    c2 = jnp.sum(cbt * cbt, axis=0, keepdims=True)

    @pl.when(i == 0)
    def _init():
        cwb_ref[...] = (
            jnp.dot(cb_ref[...], wp_ref[...], preferred_element_type=jnp.float32)
            + bp_ref[...]
        )
        loss_ref[...] = jnp.zeros_like(loss_ref)

    y = (
        jnp.dot(xo_ref[...], we_ref[...], preferred_element_type=jnp.float32)
        + be_ref[...]
    )
    s = jnp.dot(y, cbt, preferred_element_type=jnp.float32)
    score = c2 - 2.0 * s
    minv = jnp.min(score, axis=1, keepdims=True)
    iota = lax.broadcasted_iota(jnp.int32, score.shape, 1)
    idx = jnp.min(jnp.where(score == minv, iota, _K), axis=1)
    idxo_ref[0, 0, :] = idx
    x2 = jnp.sum(y * y, axis=1, keepdims=True)
    loss_ref[...] = loss_ref[...] + jnp.sum(x2 + minv)

    yt = (
        jnp.dot(xt_ref[...], wet_ref[...], preferred_element_type=jnp.float32)
        + bet_ref[...]
    )
    st = jnp.dot(yt, cbt, preferred_element_type=jnp.float32)
    scoret = c2 - 2.0 * st
    minvt = jnp.min(scoret, axis=1, keepdims=True)
    idxt = jnp.min(jnp.where(scoret == minvt, iota, _K), axis=1)
    idxt_ref[0, 0, :] = idxt


def _tc_forward(online_x, target_x, W_enc, b_enc, W_pred, b_pred, codebook,
                cbT, W_enc_t, b_enc_t):
    full = lambda shape: pl.BlockSpec(shape, lambda i: (0,) * len(shape))
    return pl.pallas_call(
        _tc_body,
        grid=(_NBLK,),
        in_specs=[
            pl.BlockSpec((_BN, _IN_DIM), lambda i: (i, 0)),
            pl.BlockSpec((_BN, _IN_DIM), lambda i: (i, 0)),
            full((_IN_DIM, _CODE_DIM)),
            full((1, _CODE_DIM)),
            full((_CODE_DIM, _CODE_DIM)),
            full((1, _CODE_DIM)),
            full((_K, _CODE_DIM)),
            full((_CODE_DIM, _K)),
            full((_IN_DIM, _CODE_DIM)),
            full((1, _CODE_DIM)),
        ],
        out_specs=[
            pl.BlockSpec((1, 1, _BN), lambda i: (i, 0, 0)),
            pl.BlockSpec((1, 1, _BN), lambda i: (i, 0, 0)),
            full((_K, _CODE_DIM)),
            full((1, 1)),
        ],
        out_shape=[
            jax.ShapeDtypeStruct((_NBLK, 1, _BN), jnp.int32),
            jax.ShapeDtypeStruct((_NBLK, 1, _BN), jnp.int32),
            jax.ShapeDtypeStruct((_K, _CODE_DIM), jnp.float32),
            jax.ShapeDtypeStruct((1, 1), jnp.float32),
        ],
        compiler_params=pltpu.CompilerParams(
            dimension_semantics=("arbitrary",),
        ),
    )(online_x, target_x, W_enc, b_enc, W_pred, b_pred, codebook, cbT,
      W_enc_t, b_enc_t)


_RING = 5


def _sc_body(cwb_hbm, cb_hbm, idxo_hbm, idxt_hbm, outq_hbm, outt_hbm,
             idxo_v, idxt_v, *bufs_and_sems):
    bo = bufs_and_sems[0:_RING]
    bt = bufs_and_sems[_RING:2 * _RING]
    so = bufs_and_sems[2 * _RING:3 * _RING]
    st = bufs_and_sems[3 * _RING:4 * _RING]
    wid = lax.axis_index("s") * 2 + lax.axis_index("c")
    base = wid * _B_PER_W
    pltpu.sync_copy(idxo_hbm.at[pl.ds(base, _B_PER_W)], idxo_v)
    pltpu.sync_copy(idxt_hbm.at[pl.ds(base, _B_PER_W)], idxt_v)

    def start(i, b):
        sl = pl.ds(i * _CHUNK, _CHUNK)
        pltpu.async_copy(cwb_hbm.at[idxo_v.at[sl]], bo[b], so[b])
        pltpu.async_copy(cb_hbm.at[idxt_v.at[sl]], bt[b], st[b])

    for b in range(_RING):
        start(b, b)

    def group(g, carry):
        for b in range(_RING):
            i = g * _RING + b
            pltpu.make_async_copy(cwb_hbm.at[pl.ds(0, _CHUNK)], bo[b], so[b]).wait()
            pltpu.make_async_copy(cb_hbm.at[pl.ds(0, _CHUNK)], bt[b], st[b]).wait()
            off = base + i * _CHUNK
            pltpu.sync_copy(bo[b], outq_hbm.at[pl.ds(off, _CHUNK)])
            pltpu.sync_copy(bt[b], outt_hbm.at[pl.ds(off, _CHUNK)])

            @pl.when(g < (_NCHUNK // _RING) - 1)
            def _refill():
                start(i + _RING, b)

        return carry

    lax.fori_loop(0, _NCHUNK // _RING, group, 0)


def _sc_gather(cwb, codebook, idxo_p, idxt_p):
    mesh = plsc.VectorSubcoreMesh(core_axis_name="c", subcore_axis_name="s")
    scratch = (
        [pltpu.VMEM((_B_PER_W,), jnp.int32)] * 2
        + [pltpu.VMEM((_CHUNK, _CODE_DIM), jnp.float32)] * (2 * _RING)
        + [pltpu.SemaphoreType.DMA] * (2 * _RING)
    )
    fn = functools.partial(
        pl.kernel,
        mesh=mesh,
        out_type=[
            jax.ShapeDtypeStruct((_PAD_N, _CODE_DIM), jnp.float32),
            jax.ShapeDtypeStruct((_PAD_N, _CODE_DIM), jnp.float32),
        ],
        scratch_types=scratch,
        compiler_params=pltpu.CompilerParams(use_tc_tiling_on_sc=False),
    )(_sc_body)
    return fn(cwb, codebook, idxo_p, idxt_p)


def kernel(online_x, target_x, W_enc, b_enc, W_pred, b_pred, codebook,
           W_enc_t, b_enc_t):
    cbT = codebook.T
    idxo3, idxt3, cwb, loss = _tc_forward(
        online_x, target_x, W_enc, b_enc.reshape(1, -1), W_pred,
        b_pred.reshape(1, -1), codebook, cbT, W_enc_t, b_enc_t.reshape(1, -1))
    pad = _PAD_N - _N
    idxo_p = jnp.concatenate([idxo3.reshape(-1), jnp.zeros((pad,), jnp.int32)])
    idxt_p = jnp.concatenate([idxt3.reshape(-1), jnp.zeros((pad,), jnp.int32)])
    outq_p, outt_p = _sc_gather(cwb, codebook, idxo_p, idxt_p)
    online_q = outq_p[:_N]
    quantized_target = outt_p[:_N]
    commit_loss = loss[0, 0] * (_COMMIT_W / (_N * _CODE_DIM))
    return (online_q, quantized_target, commit_loss)


# speedup vs baseline: 1.4891x; 1.4881x over previous
import functools

import jax
import jax.numpy as jnp
from jax import lax
from jax.experimental import pallas as pl
from jax.experimental.pallas import tpu as pltpu
from jax.experimental.pallas import tpu_sc as plsc

_N = 100000
_IN_DIM = 128
_CODE_DIM = 64
_K = 1024
_COMMIT_W = 1.0

_BN = 1000
_NBLK = _N // _BN
_NW = 32
_B_PER_W = 3136
_PAD_N = _NW * _B_PER_W
_CHUNK = 112
_NCHUNK = _B_PER_W // _CHUNK
_PARTIAL = 96


def _tc_body(xo_ref, xt_ref, we_ref, be_ref, wp_ref, bp_ref, cb_ref, cbt_ref,
             wet_ref, bet_ref, idxo_ref, idxt_ref, cwb_ref, loss_ref):
    i = pl.program_id(0)
    cbt = cbt_ref[...]
    c2 = jnp.sum(cbt * cbt, axis=0, keepdims=True)

    @pl.when(i == 0)
    def _init():
        cwb_ref[...] = (
            jnp.dot(cb_ref[...], wp_ref[...], preferred_element_type=jnp.float32)
            + bp_ref[...]
        )
        loss_ref[...] = jnp.zeros_like(loss_ref)

    y = (
        jnp.dot(xo_ref[...], we_ref[...], preferred_element_type=jnp.float32)
        + be_ref[...]
    )
    s = jnp.dot(y, cbt, preferred_element_type=jnp.float32)
    score = c2 - 2.0 * s
    minv = jnp.min(score, axis=1, keepdims=True)
    iota = lax.broadcasted_iota(jnp.int32, score.shape, 1)
    idx = jnp.min(jnp.where(score == minv, iota, _K), axis=1)
    idxo_ref[0, 0, :] = idx
    x2 = jnp.sum(y * y, axis=1, keepdims=True)
    loss_ref[...] = loss_ref[...] + jnp.sum(x2 + minv)

    yt = (
        jnp.dot(xt_ref[...], wet_ref[...], preferred_element_type=jnp.float32)
        + bet_ref[...]
    )
    st = jnp.dot(yt, cbt, preferred_element_type=jnp.float32)
    scoret = c2 - 2.0 * st
    minvt = jnp.min(scoret, axis=1, keepdims=True)
    idxt = jnp.min(jnp.where(scoret == minvt, iota, _K), axis=1)
    idxt_ref[0, 0, :] = idxt


def _tc_forward(online_x, target_x, W_enc, b_enc, W_pred, b_pred, codebook,
                cbT, W_enc_t, b_enc_t):
    full = lambda shape: pl.BlockSpec(shape, lambda i: (0,) * len(shape))
    return pl.pallas_call(
        _tc_body,
        grid=(_NBLK,),
        in_specs=[
            pl.BlockSpec((_BN, _IN_DIM), lambda i: (i, 0)),
            pl.BlockSpec((_BN, _IN_DIM), lambda i: (i, 0)),
            full((_IN_DIM, _CODE_DIM)),
            full((1, _CODE_DIM)),
            full((_CODE_DIM, _CODE_DIM)),
            full((1, _CODE_DIM)),
            full((_K, _CODE_DIM)),
            full((_CODE_DIM, _K)),
            full((_IN_DIM, _CODE_DIM)),
            full((1, _CODE_DIM)),
        ],
        out_specs=[
            pl.BlockSpec((1, 1, _BN), lambda i: (i, 0, 0)),
            pl.BlockSpec((1, 1, _BN), lambda i: (i, 0, 0)),
            full((_K, _CODE_DIM)),
            full((1, 1)),
        ],
        out_shape=[
            jax.ShapeDtypeStruct((_NBLK, 1, _BN), jnp.int32),
            jax.ShapeDtypeStruct((_NBLK, 1, _BN), jnp.int32),
            jax.ShapeDtypeStruct((_K, _CODE_DIM), jnp.float32),
            jax.ShapeDtypeStruct((1, 1), jnp.float32),
        ],
        compiler_params=pltpu.CompilerParams(
            dimension_semantics=("arbitrary",),
        ),
    )(online_x, target_x, W_enc, b_enc, W_pred, b_pred, codebook, cbT,
      W_enc_t, b_enc_t)


_RING = 4


def _sc_body(cwb_hbm, cb_hbm, idxo_hbm, idxt_hbm, outq_hbm, outt_hbm,
             idxo_v, idxt_v, cwb_sp, cb_sp, *bufs_and_sems):
    bo = bufs_and_sems[0:_RING]
    bt = bufs_and_sems[_RING:2 * _RING]
    so = bufs_and_sems[2 * _RING:3 * _RING]
    st = bufs_and_sems[3 * _RING:4 * _RING]
    sid = lax.axis_index("s")
    wid = sid * 2 + lax.axis_index("c")
    base = wid * _B_PER_W
    pltpu.sync_copy(idxo_hbm.at[pl.ds(base, _B_PER_W)], idxo_v)
    pltpu.sync_copy(idxt_hbm.at[pl.ds(base, _B_PER_W)], idxt_v)

    @pl.when(sid == 0)
    def _stage_tables():
        pltpu.sync_copy(cwb_hbm, cwb_sp)
        pltpu.sync_copy(cb_hbm, cb_sp)

    plsc.subcore_barrier()

    def start(i, b):
        sl = pl.ds(i * _CHUNK, _CHUNK)
        pltpu.async_copy(cwb_sp.at[idxo_v.at[sl]], bo[b], so[b])
        pltpu.async_copy(cb_sp.at[idxt_v.at[sl]], bt[b], st[b])

    for b in range(_RING):
        start(b, b)

    def group(g, carry):
        for b in range(_RING):
            i = g * _RING + b
            pltpu.make_async_copy(cwb_hbm.at[pl.ds(0, _CHUNK)], bo[b], so[b]).wait()
            pltpu.make_async_copy(cb_hbm.at[pl.ds(0, _CHUNK)], bt[b], st[b]).wait()
            off = base + i * _CHUNK

            @pl.when(off + _CHUNK <= _N)
            def _full_writeback():
                pltpu.sync_copy(bo[b], outq_hbm.at[pl.ds(off, _CHUNK)])
                pltpu.sync_copy(bt[b], outt_hbm.at[pl.ds(off, _CHUNK)])

            @pl.when((off < _N) & (off + _CHUNK > _N))
            def _partial_writeback():
                pltpu.sync_copy(bo[b].at[pl.ds(0, _PARTIAL)],
                                outq_hbm.at[pl.ds(_N - _PARTIAL, _PARTIAL)])
                pltpu.sync_copy(bt[b].at[pl.ds(0, _PARTIAL)],
                                outt_hbm.at[pl.ds(_N - _PARTIAL, _PARTIAL)])

            @pl.when(g < (_NCHUNK // _RING) - 1)
            def _refill():
                start(i + _RING, b)

        return carry

    lax.fori_loop(0, _NCHUNK // _RING, group, 0)


def _sc_gather(cwb, codebook, idxo_p, idxt_p):
    mesh = plsc.VectorSubcoreMesh(core_axis_name="c", subcore_axis_name="s")
    scratch = (
        [pltpu.VMEM((_B_PER_W,), jnp.int32)] * 2
        + [pltpu.VMEM_SHARED((_K, _CODE_DIM), jnp.float32)] * 2
        + [pltpu.VMEM((_CHUNK, _CODE_DIM), jnp.float32)] * (2 * _RING)
        + [pltpu.SemaphoreType.DMA] * (2 * _RING)
    )
    fn = functools.partial(
        pl.kernel,
        mesh=mesh,
        out_type=[
            jax.ShapeDtypeStruct((_N, _CODE_DIM), jnp.float32),
            jax.ShapeDtypeStruct((_N, _CODE_DIM), jnp.float32),
        ],
        scratch_types=scratch,
        compiler_params=pltpu.CompilerParams(use_tc_tiling_on_sc=False),
    )(_sc_body)
    return fn(cwb, codebook, idxo_p, idxt_p)


def kernel(online_x, target_x, W_enc, b_enc, W_pred, b_pred, codebook,
           W_enc_t, b_enc_t):
    cbT = codebook.T
    idxo3, idxt3, cwb, loss = _tc_forward(
        online_x, target_x, W_enc, b_enc.reshape(1, -1), W_pred,
        b_pred.reshape(1, -1), codebook, cbT, W_enc_t, b_enc_t.reshape(1, -1))
    pad = _PAD_N - _N
    idxo_p = jnp.concatenate([idxo3.reshape(-1), jnp.zeros((pad,), jnp.int32)])
    idxt_p = jnp.concatenate([idxt3.reshape(-1), jnp.zeros((pad,), jnp.int32)])
    online_q, quantized_target = _sc_gather(cwb, codebook, idxo_p, idxt_p)
    commit_loss = loss[0, 0] * (_COMMIT_W / (_N * _CODE_DIM))
    return (online_q, quantized_target, commit_loss)


# speedup vs baseline: 1.7094x; 1.1479x over previous
import functools

import jax
import jax.numpy as jnp
from jax import lax
from jax.experimental import pallas as pl
from jax.experimental.pallas import tpu as pltpu
from jax.experimental.pallas import tpu_sc as plsc

_N = 100000
_IN_DIM = 128
_CODE_DIM = 64
_K = 1024
_COMMIT_W = 1.0

_BN = 1000
_NBLK = _N // _BN
_NW = 32
_B_PER_W = 3136
_PAD_N = _NW * _B_PER_W
_CHUNK = 112
_NCHUNK = _B_PER_W // _CHUNK
_PARTIAL = 96


def _tc_body(xo_ref, xt_ref, we_ref, be_ref, wp_ref, bp_ref, cb_ref, cbt_ref,
             wet_ref, bet_ref, idxo_ref, idxt_ref, cwb_ref, loss_ref, c2h_ref):
    i = pl.program_id(0)
    live = i < _NBLK
    cbt = cbt_ref[...]

    @pl.when(i == 0)
    def _init():
        cwb_ref[...] = (
            jnp.dot(cb_ref[...], wp_ref[...], preferred_element_type=jnp.float32)
            + bp_ref[...]
        )
        loss_ref[...] = jnp.zeros_like(loss_ref)
        c2h_ref[...] = 0.5 * jnp.sum(cbt * cbt, axis=0, keepdims=True)

    c2h = c2h_ref[...]
    y = (
        jnp.dot(xo_ref[...], we_ref[...], preferred_element_type=jnp.float32)
        + be_ref[...]
    )
    m = jnp.dot(y, cbt, preferred_element_type=jnp.float32) - c2h
    maxv = jnp.max(m, axis=1, keepdims=True)
    iota = lax.broadcasted_iota(jnp.int32, m.shape, 1)
    idx = jnp.min(jnp.where(m == maxv, iota, _K), axis=1)
    x2 = jnp.sum(y * y, axis=1, keepdims=True)

    yt = (
        jnp.dot(xt_ref[...], wet_ref[...], preferred_element_type=jnp.float32)
        + bet_ref[...]
    )
    mt = jnp.dot(yt, cbt, preferred_element_type=jnp.float32) - c2h
    maxvt = jnp.max(mt, axis=1, keepdims=True)
    idxt = jnp.min(jnp.where(mt == maxvt, iota, _K), axis=1)

    @pl.when(live)
    def _store():
        idxo_ref[0, 0, :] = idx
        idxt_ref[0, 0, :] = idxt
        loss_ref[...] = loss_ref[...] + (jnp.sum(x2) - 2.0 * jnp.sum(maxv))

    @pl.when(jnp.logical_not(live))
    def _store_pad():
        idxo_ref[0, 0, :] = jnp.zeros((_BN,), jnp.int32)
        idxt_ref[0, 0, :] = jnp.zeros((_BN,), jnp.int32)


def _tc_forward(online_x, target_x, W_enc, b_enc, W_pred, b_pred, codebook,
                cbT, W_enc_t, b_enc_t):
    full = lambda shape: pl.BlockSpec(shape, lambda i: (0,) * len(shape))
    clamped = lambda i: (jnp.minimum(i, _NBLK - 1), 0)
    return pl.pallas_call(
        _tc_body,
        grid=(_NBLK + 1,),
        in_specs=[
            pl.BlockSpec((_BN, _IN_DIM), clamped),
            pl.BlockSpec((_BN, _IN_DIM), clamped),
            full((_IN_DIM, _CODE_DIM)),
            full((1, _CODE_DIM)),
            full((_CODE_DIM, _CODE_DIM)),
            full((1, _CODE_DIM)),
            full((_K, _CODE_DIM)),
            full((_CODE_DIM, _K)),
            full((_IN_DIM, _CODE_DIM)),
            full((1, _CODE_DIM)),
        ],
        out_specs=[
            pl.BlockSpec((1, 1, _BN), lambda i: (i, 0, 0)),
            pl.BlockSpec((1, 1, _BN), lambda i: (i, 0, 0)),
            full((_K, _CODE_DIM)),
            full((1, 1)),
        ],
        out_shape=[
            jax.ShapeDtypeStruct((_NBLK + 1, 1, _BN), jnp.int32),
            jax.ShapeDtypeStruct((_NBLK + 1, 1, _BN), jnp.int32),
            jax.ShapeDtypeStruct((_K, _CODE_DIM), jnp.float32),
            jax.ShapeDtypeStruct((1, 1), jnp.float32),
        ],
        scratch_shapes=[pltpu.VMEM((1, _K), jnp.float32)],
        compiler_params=pltpu.CompilerParams(
            dimension_semantics=("arbitrary",),
        ),
    )(online_x, target_x, W_enc, b_enc, W_pred, b_pred, codebook, cbT,
      W_enc_t, b_enc_t)


_RING = 4


def _sc_body(cwb_hbm, cb_hbm, idxo_hbm, idxt_hbm, outq_hbm, outt_hbm,
             idxo_v, idxt_v, cwb_sp, cb_sp, *bufs_and_sems):
    bo = bufs_and_sems[0:_RING]
    bt = bufs_and_sems[_RING:2 * _RING]
    so = bufs_and_sems[2 * _RING:3 * _RING]
    st = bufs_and_sems[3 * _RING:4 * _RING]
    sid = lax.axis_index("s")
    wid = sid * 2 + lax.axis_index("c")
    base = wid * _B_PER_W
    pltpu.sync_copy(idxo_hbm.at[pl.ds(base, _B_PER_W)], idxo_v)
    pltpu.sync_copy(idxt_hbm.at[pl.ds(base, _B_PER_W)], idxt_v)

    @pl.when(sid == 0)
    def _stage_tables():
        pltpu.sync_copy(cwb_hbm, cwb_sp)
        pltpu.sync_copy(cb_hbm, cb_sp)

    plsc.subcore_barrier()

    def start(i, b):
        sl = pl.ds(i * _CHUNK, _CHUNK)
        pltpu.async_copy(cwb_sp.at[idxo_v.at[sl]], bo[b], so[b])
        pltpu.async_copy(cb_sp.at[idxt_v.at[sl]], bt[b], st[b])

    for b in range(_RING):
        start(b, b)

    def group(g, carry):
        for b in range(_RING):
            i = g * _RING + b
            pltpu.make_async_copy(cwb_hbm.at[pl.ds(0, _CHUNK)], bo[b], so[b]).wait()
            pltpu.make_async_copy(cb_hbm.at[pl.ds(0, _CHUNK)], bt[b], st[b]).wait()
            off = base + i * _CHUNK

            @pl.when(off + _CHUNK <= _N)
            def _full_writeback():
                pltpu.sync_copy(bo[b], outq_hbm.at[pl.ds(off, _CHUNK)])
                pltpu.sync_copy(bt[b], outt_hbm.at[pl.ds(off, _CHUNK)])

            @pl.when((off < _N) & (off + _CHUNK > _N))
            def _partial_writeback():
                pltpu.sync_copy(bo[b].at[pl.ds(0, _PARTIAL)],
                                outq_hbm.at[pl.ds(_N - _PARTIAL, _PARTIAL)])
                pltpu.sync_copy(bt[b].at[pl.ds(0, _PARTIAL)],
                                outt_hbm.at[pl.ds(_N - _PARTIAL, _PARTIAL)])

            @pl.when(g < (_NCHUNK // _RING) - 1)
            def _refill():
                start(i + _RING, b)

        return carry

    lax.fori_loop(0, _NCHUNK // _RING, group, 0)


def _sc_gather(cwb, codebook, idxo_p, idxt_p):
    mesh = plsc.VectorSubcoreMesh(core_axis_name="c", subcore_axis_name="s")
    scratch = (
        [pltpu.VMEM((_B_PER_W,), jnp.int32)] * 2
        + [pltpu.VMEM_SHARED((_K, _CODE_DIM), jnp.float32)] * 2
        + [pltpu.VMEM((_CHUNK, _CODE_DIM), jnp.float32)] * (2 * _RING)
        + [pltpu.SemaphoreType.DMA] * (2 * _RING)
    )
    fn = functools.partial(
        pl.kernel,
        mesh=mesh,
        out_type=[
            jax.ShapeDtypeStruct((_N, _CODE_DIM), jnp.float32),
            jax.ShapeDtypeStruct((_N, _CODE_DIM), jnp.float32),
        ],
        scratch_types=scratch,
        compiler_params=pltpu.CompilerParams(use_tc_tiling_on_sc=False),
    )(_sc_body)
    return fn(cwb, codebook, idxo_p, idxt_p)


def kernel(online_x, target_x, W_enc, b_enc, W_pred, b_pred, codebook,
           W_enc_t, b_enc_t):
    cbT = codebook.T
    idxo3, idxt3, cwb, loss = _tc_forward(
        online_x, target_x, W_enc, b_enc.reshape(1, -1), W_pred,
        b_pred.reshape(1, -1), codebook, cbT, W_enc_t, b_enc_t.reshape(1, -1))
    online_q, quantized_target = _sc_gather(
        cwb, codebook, idxo3.reshape(-1), idxt3.reshape(-1))
    commit_loss = loss[0, 0] * (_COMMIT_W / (_N * _CODE_DIM))
    return (online_q, quantized_target, commit_loss)


# speedup vs baseline: 1.8861x; 1.1034x over previous
import functools

import jax
import jax.numpy as jnp
from jax import lax
from jax.experimental import pallas as pl
from jax.experimental.pallas import tpu as pltpu
from jax.experimental.pallas import tpu_sc as plsc

_N = 100000
_IN_DIM = 128
_CODE_DIM = 64
_K = 1024
_COMMIT_W = 1.0

_BN = 1000
_NBLK = _N // _BN
_NW = 32
_B_PER_W = 3136
_PAD_N = _NW * _B_PER_W
_CHUNK = 112
_NCHUNK = _B_PER_W // _CHUNK
_PARTIAL = 96


def _tc_body(xo_ref, xt_ref, we_ref, be_ref, wp_ref, bp_ref, cb_ref, cbt_ref,
             wet_ref, bet_ref, idxo_ref, idxt_ref, cwb_ref, loss_ref,
             c2h_ref, iotaf_ref):
    i = pl.program_id(0)
    live = i < _NBLK
    cbt = cbt_ref[...]

    @pl.when(i == 0)
    def _init():
        cwb_ref[...] = (
            jnp.dot(cb_ref[...], wp_ref[...], preferred_element_type=jnp.float32)
            + bp_ref[...]
        )
        loss_ref[...] = jnp.zeros_like(loss_ref)
        c2h_ref[...] = 0.5 * jnp.sum(cbt * cbt, axis=0, keepdims=True)
        iotaf_ref[...] = lax.broadcasted_iota(
            jnp.int32, (1, _K), 1).astype(jnp.float32)

    c2h = c2h_ref[...]
    y = (
        jnp.dot(xo_ref[...], we_ref[...], preferred_element_type=jnp.float32)
        + be_ref[...]
    )
    m = jnp.dot(y, cbt, preferred_element_type=jnp.float32) - c2h
    maxv = jnp.max(m, axis=1, keepdims=True)
    iota = iotaf_ref[...]
    idx = jnp.min(jnp.where(m == maxv, iota, float(_K)),
                  axis=1).astype(jnp.int32)
    x2 = jnp.sum(y * y, axis=1, keepdims=True)

    yt = (
        jnp.dot(xt_ref[...], wet_ref[...], preferred_element_type=jnp.float32)
        + bet_ref[...]
    )
    mt = jnp.dot(yt, cbt, preferred_element_type=jnp.float32) - c2h
    maxvt = jnp.max(mt, axis=1, keepdims=True)
    idxt = jnp.min(jnp.where(mt == maxvt, iota, float(_K)),
                   axis=1).astype(jnp.int32)

    @pl.when(live)
    def _store():
        idxo_ref[0, 0, :] = idx
        idxt_ref[0, 0, :] = idxt
        loss_ref[...] = loss_ref[...] + (jnp.sum(x2) - 2.0 * jnp.sum(maxv))

    @pl.when(jnp.logical_not(live))
    def _store_pad():
        idxo_ref[0, 0, :] = jnp.zeros((_BN,), jnp.int32)
        idxt_ref[0, 0, :] = jnp.zeros((_BN,), jnp.int32)


def _tc_forward(online_x, target_x, W_enc, b_enc, W_pred, b_pred, codebook,
                cbT, W_enc_t, b_enc_t):
    full = lambda shape: pl.BlockSpec(shape, lambda i: (0,) * len(shape))
    clamped = lambda i: (jnp.minimum(i, _NBLK - 1), 0)
    return pl.pallas_call(
        _tc_body,
        grid=(_NBLK + 1,),
        in_specs=[
            pl.BlockSpec((_BN, _IN_DIM), clamped),
            pl.BlockSpec((_BN, _IN_DIM), clamped),
            full((_IN_DIM, _CODE_DIM)),
            full((1, _CODE_DIM)),
            full((_CODE_DIM, _CODE_DIM)),
            full((1, _CODE_DIM)),
            full((_K, _CODE_DIM)),
            full((_CODE_DIM, _K)),
            full((_IN_DIM, _CODE_DIM)),
            full((1, _CODE_DIM)),
        ],
        out_specs=[
            pl.BlockSpec((1, 1, _BN), lambda i: (i, 0, 0)),
            pl.BlockSpec((1, 1, _BN), lambda i: (i, 0, 0)),
            full((_K, _CODE_DIM)),
            full((1, 1)),
        ],
        out_shape=[
            jax.ShapeDtypeStruct((_NBLK + 1, 1, _BN), jnp.int32),
            jax.ShapeDtypeStruct((_NBLK + 1, 1, _BN), jnp.int32),
            jax.ShapeDtypeStruct((_K, _CODE_DIM), jnp.float32),
            jax.ShapeDtypeStruct((1, 1), jnp.float32),
        ],
        scratch_shapes=[pltpu.VMEM((1, _K), jnp.float32),
                        pltpu.VMEM((1, _K), jnp.float32)],
        compiler_params=pltpu.CompilerParams(
            dimension_semantics=("arbitrary",),
        ),
    )(online_x, target_x, W_enc, b_enc, W_pred, b_pred, codebook, cbT,
      W_enc_t, b_enc_t)


_RING = 4


def _sc_body(cwb_hbm, cb_hbm, idxo_hbm, idxt_hbm, outq_hbm, outt_hbm,
             idxo_v, idxt_v, cwb_sp, cb_sp, *bufs_and_sems):
    bo = bufs_and_sems[0:_RING]
    bt = bufs_and_sems[_RING:2 * _RING]
    so = bufs_and_sems[2 * _RING:3 * _RING]
    st = bufs_and_sems[3 * _RING:4 * _RING]
    sid = lax.axis_index("s")
    wid = sid * 2 + lax.axis_index("c")
    base = wid * _B_PER_W
    pltpu.sync_copy(idxo_hbm.at[pl.ds(base, _B_PER_W)], idxo_v)
    pltpu.sync_copy(idxt_hbm.at[pl.ds(base, _B_PER_W)], idxt_v)

    @pl.when(sid == 0)
    def _stage_tables():
        pltpu.sync_copy(cwb_hbm, cwb_sp)
        pltpu.sync_copy(cb_hbm, cb_sp)

    plsc.subcore_barrier()

    def start(i, b):
        sl = pl.ds(i * _CHUNK, _CHUNK)
        pltpu.async_copy(cwb_sp.at[idxo_v.at[sl]], bo[b], so[b])
        pltpu.async_copy(cb_sp.at[idxt_v.at[sl]], bt[b], st[b])

    for b in range(_RING):
        start(b, b)

    def group(g, carry):
        for b in range(_RING):
            i = g * _RING + b
            pltpu.make_async_copy(cwb_hbm.at[pl.ds(0, _CHUNK)], bo[b], so[b]).wait()
            pltpu.make_async_copy(cb_hbm.at[pl.ds(0, _CHUNK)], bt[b], st[b]).wait()
            off = base + i * _CHUNK

            @pl.when(off + _CHUNK <= _N)
            def _full_writeback():
                pltpu.sync_copy(bo[b], outq_hbm.at[pl.ds(off, _CHUNK)])
                pltpu.sync_copy(bt[b], outt_hbm.at[pl.ds(off, _CHUNK)])

            @pl.when((off < _N) & (off + _CHUNK > _N))
            def _partial_writeback():
                pltpu.sync_copy(bo[b].at[pl.ds(0, _PARTIAL)],
                                outq_hbm.at[pl.ds(_N - _PARTIAL, _PARTIAL)])
                pltpu.sync_copy(bt[b].at[pl.ds(0, _PARTIAL)],
                                outt_hbm.at[pl.ds(_N - _PARTIAL, _PARTIAL)])

            @pl.when(g < (_NCHUNK // _RING) - 1)
            def _refill():
                start(i + _RING, b)

        return carry

    lax.fori_loop(0, _NCHUNK // _RING, group, 0)


def _sc_gather(cwb, codebook, idxo_p, idxt_p):
    mesh = plsc.VectorSubcoreMesh(core_axis_name="c", subcore_axis_name="s")
    scratch = (
        [pltpu.VMEM((_B_PER_W,), jnp.int32)] * 2
        + [pltpu.VMEM_SHARED((_K, _CODE_DIM), jnp.float32)] * 2
        + [pltpu.VMEM((_CHUNK, _CODE_DIM), jnp.float32)] * (2 * _RING)
        + [pltpu.SemaphoreType.DMA] * (2 * _RING)
    )
    fn = functools.partial(
        pl.kernel,
        mesh=mesh,
        out_type=[
            jax.ShapeDtypeStruct((_N, _CODE_DIM), jnp.float32),
            jax.ShapeDtypeStruct((_N, _CODE_DIM), jnp.float32),
        ],
        scratch_types=scratch,
        compiler_params=pltpu.CompilerParams(use_tc_tiling_on_sc=False),
    )(_sc_body)
    return fn(cwb, codebook, idxo_p, idxt_p)


def kernel(online_x, target_x, W_enc, b_enc, W_pred, b_pred, codebook,
           W_enc_t, b_enc_t):
    cbT = codebook.T
    idxo3, idxt3, cwb, loss = _tc_forward(
        online_x, target_x, W_enc, b_enc.reshape(1, -1), W_pred,
        b_pred.reshape(1, -1), codebook, cbT, W_enc_t, b_enc_t.reshape(1, -1))
    online_q, quantized_target = _sc_gather(
        cwb, codebook, idxo3.reshape(-1), idxt3.reshape(-1))
    commit_loss = loss[0, 0] * (_COMMIT_W / (_N * _CODE_DIM))
    return (online_q, quantized_target, commit_loss)


# speedup vs baseline: 2.2688x; 1.2029x over previous
import functools

import jax
import jax.numpy as jnp
from jax import lax
from jax.experimental import pallas as pl
from jax.experimental.pallas import tpu as pltpu
from jax.experimental.pallas import tpu_sc as plsc

_N = 100000
_IN_DIM = 128
_CODE_DIM = 64
_K = 1024
_COMMIT_W = 1.0

_BN = 1024
_NBLK = 98
_NW = 32
_B_PER_W = 3136
_PAD_N = _NW * _B_PER_W
_CHUNK = 112
_NCHUNK = _B_PER_W // _CHUNK
_PARTIAL = 96


def _tc_body(xo_ref, xt_ref, we_ref, be_ref, wp_ref, bp_ref, cb_ref, cbt_ref,
             wet_ref, bet_ref, idxo_ref, idxt_ref, cwb_ref, loss_ref,
             c2h_ref, iotaf_ref):
    i = pl.program_id(0)
    cbt = cbt_ref[...]

    @pl.when(i == 0)
    def _init():
        cwb_ref[...] = (
            jnp.dot(cb_ref[...], wp_ref[...], preferred_element_type=jnp.float32)
            + bp_ref[...]
        )
        loss_ref[...] = jnp.zeros_like(loss_ref)
        c2h_ref[...] = 0.5 * jnp.sum(cbt * cbt, axis=0, keepdims=True)
        iotaf_ref[...] = lax.broadcasted_iota(
            jnp.int32, (1, _K), 1).astype(jnp.float32)

    c2h = c2h_ref[...]
    y = (
        jnp.dot(xo_ref[...], we_ref[...], preferred_element_type=jnp.float32)
        + be_ref[...]
    )
    m = jnp.dot(y, cbt, preferred_element_type=jnp.float32) - c2h
    maxv = jnp.max(m, axis=1, keepdims=True)
    iota = iotaf_ref[...]
    idx = jnp.minimum(jnp.min(jnp.where(m == maxv, iota, float(_K)), axis=1),
                      float(_K - 1)).astype(jnp.int32)
    x2 = jnp.sum(y * y, axis=1, keepdims=True)

    yt = (
        jnp.dot(xt_ref[...], wet_ref[...], preferred_element_type=jnp.float32)
        + bet_ref[...]
    )
    mt = jnp.dot(yt, cbt, preferred_element_type=jnp.float32) - c2h
    maxvt = jnp.max(mt, axis=1, keepdims=True)
    idxt = jnp.minimum(jnp.min(jnp.where(mt == maxvt, iota, float(_K)), axis=1),
                       float(_K - 1)).astype(jnp.int32)

    for r in range(8):
        idxo_ref[0, r, :] = lax.slice(idx, (r * 128,), ((r + 1) * 128,))
        idxt_ref[0, r, :] = lax.slice(idxt, (r * 128,), ((r + 1) * 128,))
    valid_col = (i * _BN + lax.broadcasted_iota(jnp.int32, (_BN, 1), 0)) < _N
    loss_ref[...] = loss_ref[...] + jnp.sum(
        jnp.where(valid_col, x2 - 2.0 * maxv, 0.0))


def _tc_forward(online_x, target_x, W_enc, b_enc, W_pred, b_pred, codebook,
                cbT, W_enc_t, b_enc_t):
    full = lambda shape: pl.BlockSpec(shape, lambda i: (0,) * len(shape))
    return pl.pallas_call(
        _tc_body,
        grid=(_NBLK,),
        in_specs=[
            pl.BlockSpec((_BN, _IN_DIM), lambda i: (i, 0)),
            pl.BlockSpec((_BN, _IN_DIM), lambda i: (i, 0)),
            full((_IN_DIM, _CODE_DIM)),
            full((1, _CODE_DIM)),
            full((_CODE_DIM, _CODE_DIM)),
            full((1, _CODE_DIM)),
            full((_K, _CODE_DIM)),
            full((_CODE_DIM, _K)),
            full((_IN_DIM, _CODE_DIM)),
            full((1, _CODE_DIM)),
        ],
        out_specs=[
            pl.BlockSpec((1, 8, 128), lambda i: (i, 0, 0)),
            pl.BlockSpec((1, 8, 128), lambda i: (i, 0, 0)),
            full((_K, _CODE_DIM)),
            full((1, 1)),
        ],
        out_shape=[
            jax.ShapeDtypeStruct((_NBLK, 8, 128), jnp.int32),
            jax.ShapeDtypeStruct((_NBLK, 8, 128), jnp.int32),
            jax.ShapeDtypeStruct((_K, _CODE_DIM), jnp.float32),
            jax.ShapeDtypeStruct((1, 1), jnp.float32),
        ],
        scratch_shapes=[pltpu.VMEM((1, _K), jnp.float32),
                        pltpu.VMEM((1, _K), jnp.float32)],
        compiler_params=pltpu.CompilerParams(
            dimension_semantics=("arbitrary",),
        ),
    )(online_x, target_x, W_enc, b_enc, W_pred, b_pred, codebook, cbT,
      W_enc_t, b_enc_t)


_RING = 4


def _sc_body(cwb_hbm, cb_hbm, idxo_hbm, idxt_hbm, outq_hbm, outt_hbm,
             idxo_v, idxt_v, cwb_sp, cb_sp, *bufs_and_sems):
    bo = bufs_and_sems[0:_RING]
    bt = bufs_and_sems[_RING:2 * _RING]
    so = bufs_and_sems[2 * _RING:3 * _RING]
    st = bufs_and_sems[3 * _RING:4 * _RING]
    sid = lax.axis_index("s")
    wid = sid * 2 + lax.axis_index("c")
    base = wid * _B_PER_W
    pltpu.sync_copy(idxo_hbm.at[pl.ds(base, _B_PER_W)], idxo_v)
    pltpu.sync_copy(idxt_hbm.at[pl.ds(base, _B_PER_W)], idxt_v)

    @pl.when(sid == 0)
    def _stage_tables():
        pltpu.sync_copy(cwb_hbm, cwb_sp)
        pltpu.sync_copy(cb_hbm, cb_sp)

    plsc.subcore_barrier()

    def start(i, b):
        sl = pl.ds(i * _CHUNK, _CHUNK)
        pltpu.async_copy(cwb_sp.at[idxo_v.at[sl]], bo[b], so[b])
        pltpu.async_copy(cb_sp.at[idxt_v.at[sl]], bt[b], st[b])

    for b in range(_RING):
        start(b, b)

    def group(g, carry):
        for b in range(_RING):
            i = g * _RING + b
            pltpu.make_async_copy(cwb_hbm.at[pl.ds(0, _CHUNK)], bo[b], so[b]).wait()
            pltpu.make_async_copy(cb_hbm.at[pl.ds(0, _CHUNK)], bt[b], st[b]).wait()
            off = base + i * _CHUNK

            @pl.when(off + _CHUNK <= _N)
            def _full_writeback():
                pltpu.sync_copy(bo[b], outq_hbm.at[pl.ds(off, _CHUNK)])
                pltpu.sync_copy(bt[b], outt_hbm.at[pl.ds(off, _CHUNK)])

            @pl.when((off < _N) & (off + _CHUNK > _N))
            def _partial_writeback():
                pltpu.sync_copy(bo[b].at[pl.ds(0, _PARTIAL)],
                                outq_hbm.at[pl.ds(_N - _PARTIAL, _PARTIAL)])
                pltpu.sync_copy(bt[b].at[pl.ds(0, _PARTIAL)],
                                outt_hbm.at[pl.ds(_N - _PARTIAL, _PARTIAL)])

            @pl.when(g < (_NCHUNK // _RING) - 1)
            def _refill():
                start(i + _RING, b)

        return carry

    lax.fori_loop(0, _NCHUNK // _RING, group, 0)


def _sc_gather(cwb, codebook, idxo_p, idxt_p):
    mesh = plsc.VectorSubcoreMesh(core_axis_name="c", subcore_axis_name="s")
    scratch = (
        [pltpu.VMEM((_B_PER_W,), jnp.int32)] * 2
        + [pltpu.VMEM_SHARED((_K, _CODE_DIM), jnp.float32)] * 2
        + [pltpu.VMEM((_CHUNK, _CODE_DIM), jnp.float32)] * (2 * _RING)
        + [pltpu.SemaphoreType.DMA] * (2 * _RING)
    )
    fn = functools.partial(
        pl.kernel,
        mesh=mesh,
        out_type=[
            jax.ShapeDtypeStruct((_N, _CODE_DIM), jnp.float32),
            jax.ShapeDtypeStruct((_N, _CODE_DIM), jnp.float32),
        ],
        scratch_types=scratch,
        compiler_params=pltpu.CompilerParams(use_tc_tiling_on_sc=False),
    )(_sc_body)
    return fn(cwb, codebook, idxo_p, idxt_p)


def kernel(online_x, target_x, W_enc, b_enc, W_pred, b_pred, codebook,
           W_enc_t, b_enc_t):
    cbT = codebook.T
    idxo3, idxt3, cwb, loss = _tc_forward(
        online_x, target_x, W_enc, b_enc.reshape(1, -1), W_pred,
        b_pred.reshape(1, -1), codebook, cbT, W_enc_t, b_enc_t.reshape(1, -1))
    online_q, quantized_target = _sc_gather(
        cwb, codebook, idxo3.reshape(-1), idxt3.reshape(-1))
    commit_loss = loss[0, 0] * (_COMMIT_W / (_N * _CODE_DIM))
    return (online_q, quantized_target, commit_loss)
